# baseline (speedup 1.0000x reference)
import functools

import jax
import jax.numpy as jnp
from jax import lax
from jax.experimental import pallas as pl
from jax.experimental.pallas import tpu as pltpu
from jax.experimental.pallas import tpu_sc as plsc

HIDDEN_DIM = 128
VOCAB_ROWS = 118
N = 100000

_NC = 2
_NS = 16
_NW = _NC * _NS

_PER_W = 3136
_CHUNK = 392
_NCHUNK = _PER_W // _CHUNK
_PER_W_LAST = N - (_NW - 1) * _PER_W
_TAIL = _PER_W_LAST - (_NCHUNK - 1) * _CHUNK
_HROWS = 24


@functools.partial(
    pl.kernel,
    mesh=plsc.VectorSubcoreMesh(core_axis_name="c", subcore_axis_name="s"),
    out_type=jax.ShapeDtypeStruct((N, HIDDEN_DIM), jnp.float32),
    scratch_types=[
        pltpu.VMEM((_PER_W,), jnp.int32),
        pltpu.VMEM((_CHUNK, HIDDEN_DIM), jnp.float32),
        pltpu.VMEM((_CHUNK, HIDDEN_DIM), jnp.float32),
        pltpu.VMEM_SHARED((VOCAB_ROWS, HIDDEN_DIM), jnp.float32),
        pltpu.SemaphoreType.DMA,
        pltpu.SemaphoreType.DMA,
        pltpu.SemaphoreType.DMA,
        pltpu.SemaphoreType.DMA,
        pltpu.SemaphoreType.DMA,
        pltpu.SemaphoreType.DMA,
        pltpu.SemaphoreType.DMA,
    ],
)
def _embedding_gather(table_hbm, idx_hbm, out_hbm, idx_all, rows0, rows1,
                      table_sh, tsem, gsem0, gsem1, hsem0, hsem1,
                      osem0, osem1):
    wid = lax.axis_index("s") * _NC + lax.axis_index("c")
    base = wid * _PER_W
    rows = (rows0, rows1)
    gsem = (gsem0, gsem1)
    hsem = (hsem0, hsem1)
    osem = (osem0, osem1)

    sid = lax.axis_index("s")
    tl = pltpu.make_async_copy(table_hbm, table_sh, tsem)

    @pl.when(sid == 0)
    def _():
        tl.start()

    @pl.when(wid < _NW - 1)
    def _():
        pltpu.sync_copy(idx_hbm.at[pl.ds(base, _PER_W)], idx_all)

    @pl.when(wid == _NW - 1)
    def _():
        pltpu.sync_copy(idx_hbm.at[pl.ds(base, _PER_W_LAST)],
                        idx_all.at[pl.ds(0, _PER_W_LAST)])

    @pl.when(sid == 0)
    def _():
        tl.wait()

    plsc.subcore_barrier()

    def chunk(k, nrows, b):
        if nrows == _CHUNK:
            gh = pltpu.async_copy(
                table_hbm.at[idx_all.at[pl.ds(k * _CHUNK, _HROWS)]],
                rows[b].at[pl.ds(0, _HROWS)], hsem[b])
            gs = pltpu.async_copy(
                table_sh.at[idx_all.at[pl.ds(k * _CHUNK + _HROWS,
                                             _CHUNK - _HROWS)]],
                rows[b].at[pl.ds(_HROWS, _CHUNK - _HROWS)], gsem[b])
            gs.wait()
            gh.wait()
        else:
            pltpu.async_copy(
                table_sh.at[idx_all.at[pl.ds(k * _CHUNK, nrows)]],
                rows[b].at[pl.ds(0, nrows)], gsem[b]).wait()
        return pltpu.async_copy(
            rows[b].at[pl.ds(0, nrows)],
            out_hbm.at[pl.ds(base + k * _CHUNK, nrows)], osem[b])

    stores = [None, None]
    for k in range(_NCHUNK - 1):
        b = k & 1
        if stores[b] is not None:
            stores[b].wait()
        stores[b] = chunk(k, _CHUNK, b)

    stores[1].wait()

    @pl.when(wid < _NW - 1)
    def _():
        chunk(_NCHUNK - 1, _CHUNK, 1).wait()

    @pl.when(wid == _NW - 1)
    def _():
        chunk(_NCHUNK - 1, _TAIL, 1).wait()

    stores[0].wait()


def kernel(atom_num, table):
    idx = atom_num.astype(jnp.int32)
    return _embedding_gather(table, idx)


# speedup vs baseline: 1.1748x; 1.1748x over previous
import functools

import jax
import jax.numpy as jnp
from jax import lax
from jax.experimental import pallas as pl
from jax.experimental.pallas import tpu as pltpu
from jax.experimental.pallas import tpu_sc as plsc

HIDDEN_DIM = 128
VOCAB_ROWS = 118
N = 100000

_NC = 2
_NS = 16
_NW = _NC * _NS

_PER_W = 3136
_CHUNK = 392
_NCHUNK = _PER_W // _CHUNK
_PER_W_LAST = N - (_NW - 1) * _PER_W
_TAIL = _PER_W_LAST - (_NCHUNK - 1) * _CHUNK


@functools.partial(
    pl.kernel,
    mesh=plsc.VectorSubcoreMesh(core_axis_name="c", subcore_axis_name="s"),
    out_type=jax.ShapeDtypeStruct((N, HIDDEN_DIM), jnp.float32),
    scratch_types=[
        pltpu.VMEM((_PER_W,), jnp.int32),
        pltpu.VMEM((_CHUNK, HIDDEN_DIM), jnp.float32),
        pltpu.VMEM((_CHUNK, HIDDEN_DIM), jnp.float32),
        pltpu.VMEM_SHARED((VOCAB_ROWS, HIDDEN_DIM), jnp.float32),
        pltpu.SemaphoreType.DMA,
        pltpu.SemaphoreType.DMA,
        pltpu.SemaphoreType.DMA,
        pltpu.SemaphoreType.DMA,
        pltpu.SemaphoreType.DMA,
    ],
)
def _embedding_gather(table_hbm, idx_hbm, out_hbm, idx_all, rows0, rows1,
                      table_sh, tsem, gsem0, gsem1, osem0, osem1):
    wid = lax.axis_index("s") * _NC + lax.axis_index("c")
    base = wid * _PER_W
    rows = (rows0, rows1)
    gsem = (gsem0, gsem1)
    osem = (osem0, osem1)

    sid = lax.axis_index("s")
    tl = pltpu.make_async_copy(table_hbm, table_sh, tsem)

    @pl.when(sid == 0)
    def _():
        tl.start()

    @pl.when(wid < _NW - 1)
    def _():
        pltpu.sync_copy(idx_hbm.at[pl.ds(base, _PER_W)], idx_all)

    @pl.when(wid == _NW - 1)
    def _():
        pltpu.sync_copy(idx_hbm.at[pl.ds(base, _PER_W_LAST)],
                        idx_all.at[pl.ds(0, _PER_W_LAST)])

    @pl.when(sid == 0)
    def _():
        tl.wait()

    plsc.subcore_barrier()

    def chunk(k, nrows, b):
        pltpu.async_copy(
            table_sh.at[idx_all.at[pl.ds(k * _CHUNK, nrows)]],
            rows[b].at[pl.ds(0, nrows)], gsem[b]).wait()
        return pltpu.async_copy(
            rows[b].at[pl.ds(0, nrows)],
            out_hbm.at[pl.ds(base + k * _CHUNK, nrows)], osem[b])

    stores = [None, None]
    for k in range(_NCHUNK - 1):
        b = k & 1
        if stores[b] is not None:
            stores[b].wait()
        stores[b] = chunk(k, _CHUNK, b)

    stores[1].wait()

    @pl.when(wid < _NW - 1)
    def _():
        chunk(_NCHUNK - 1, _CHUNK, 1).wait()

    @pl.when(wid == _NW - 1)
    def _():
        chunk(_NCHUNK - 1, _TAIL, 1).wait()

    stores[0].wait()


def kernel(atom_num, table):
    idx = atom_num.astype(jnp.int32)
    return _embedding_gather(table, idx)
